# baseline (speedup 1.0000x reference)
import functools

import jax
import jax.numpy as jnp
from jax import lax
from jax.experimental import pallas as pl
from jax.experimental.pallas import tpu as pltpu
from jax.experimental.pallas import tpu_sc as plsc

_FM = 128
_NV = 10000
_NCON = 10000
_NE = 320000
_OUT = 16

_NSC = 2
_NSUB = 16
_EPS = _NE // _NSUB
_CH = 64
_NCHUNK = 314
_EPSP = _NCHUNK * _CH
_RPC = 5120
_NPAD = _NSC * _RPC
_RPS = _RPC // _NSUB

_DNUMS = lax.GatherDimensionNumbers(
    offset_dims=(), collapsed_slice_dims=(0,), start_index_map=(0,))


def _segsum_body(table, gs, vals, cnts, zeros, out,
                 gs_v, vals_v, cnt_v, gbuf0, gbuf1, sloc0, sloc1,
                 grows0, grows1, srows0, srows1, acc,
                 gsem0, gsem1, ssem0, ssem1):
    c = lax.axis_index("c")
    s = lax.axis_index("s")
    base = c * _RPC
    gbuf = (gbuf0, gbuf1)
    sloc = (sloc0, sloc1)
    grows = (grows0, grows1)
    srows = (srows0, srows1)
    gsem = (gsem0, gsem1)
    ssem = (ssem0, ssem1)

    pltpu.sync_copy(zeros, acc.at[pl.ds(s * _RPS, _RPS)])
    pltpu.sync_copy(gs.at[c, s], gs_v)
    pltpu.sync_copy(vals.at[c, s], vals_v)
    pltpu.sync_copy(cnts.at[c, s], cnt_v)
    nch = cnt_v[pl.ds(0, 16)][0]

    def unpack_to(tt, gb):
        def up(g, carry):
            w16 = gs_v[pl.ds(tt * _CH + g * 16, 16)]
            gb[pl.ds(g * 16, 16)] = jnp.bitwise_and(w16, 0x3FFF)
            return carry
        lax.fori_loop(0, _CH // 16, up, 0)

    unpack_to(0, gbuf0)
    pltpu.async_copy(table.at[gbuf0], grows0, gsem0)
    pltpu.async_copy(zeros.at[pl.ds(0, _CH)], srows0, ssem0)
    pltpu.async_copy(zeros.at[pl.ds(0, _CH)], srows1, ssem1)
    plsc.subcore_barrier()

    def step(t, p):
        q = 1 - p
        tn = jnp.minimum(t + 1, _NCHUNK - 1)

        live_n = t + 1 < nch

        @pl.when(live_n)
        def _prefetch():
            unpack_to(tn, gbuf[q])

        @pl.when(t < nch)
        def _work():
            pltpu.make_async_copy(table.at[pl.ds(0, _CH)], grows[p],
                                  gsem[p]).wait()

            @pl.when(live_n)
            def _g():
                pltpu.async_copy(table.at[gbuf[q]], grows[q], gsem[q])

            pltpu.make_async_copy(zeros.at[pl.ds(0, _CH)], srows[p],
                                  ssem[p]).wait()

            def grp(g, carry2):
                w16 = gs_v[pl.ds(t * _CH + g * 16, 16)]
                sloc[p][pl.ds(g * 16, 16)] = (
                    lax.shift_right_logical(w16, 14) - base)
                v16 = vals_v[pl.ds(t * _CH + g * 16, 16)]
                for j in range(16):
                    b16 = lax.gather(
                        v16, jnp.full((16, 1), j, jnp.int32), _DNUMS, (1,),
                        mode=lax.GatherScatterMode.PROMISE_IN_BOUNDS)
                    r = g * 16 + j
                    for k in range(_FM // 16):
                        srows[p][r, pl.ds(k * 16, 16)] = (
                            grows[p][r, pl.ds(k * 16, 16)] * b16)
                return carry2

            lax.fori_loop(0, _CH // 16, grp, 0)
            pltpu.async_copy(srows[p], acc.at[sloc[p]], ssem[p], add=True)

    def duo(t2, carry):
        step(t2 * 2, 0)
        step(t2 * 2 + 1, 1)
        return carry

    lax.fori_loop(0, _NCHUNK // 2, duo, 0)
    pltpu.make_async_copy(zeros.at[pl.ds(0, _CH)], srows0, ssem0).wait()
    pltpu.make_async_copy(zeros.at[pl.ds(0, _CH)], srows1, ssem1).wait()
    plsc.subcore_barrier()
    pltpu.sync_copy(acc.at[pl.ds(s * _RPS, _RPS)],
                    out.at[pl.ds(base + s * _RPS, _RPS)])


_segsum_sc = functools.partial(
    pl.kernel,
    mesh=plsc.VectorSubcoreMesh(core_axis_name="c", subcore_axis_name="s"),
    out_type=jax.ShapeDtypeStruct((_NPAD, _FM), jnp.float32),
    scratch_types=[
        pltpu.VMEM((_EPSP,), jnp.int32),
        pltpu.VMEM((_EPSP,), jnp.float32),
        pltpu.VMEM((16,), jnp.int32),
        pltpu.VMEM((_CH,), jnp.int32),
        pltpu.VMEM((_CH,), jnp.int32),
        pltpu.VMEM((_CH,), jnp.int32),
        pltpu.VMEM((_CH,), jnp.int32),
        pltpu.VMEM((_CH, _FM), jnp.float32),
        pltpu.VMEM((_CH, _FM), jnp.float32),
        pltpu.VMEM((_CH, _FM), jnp.float32),
        pltpu.VMEM((_CH, _FM), jnp.float32),
        pltpu.VMEM_SHARED((_RPC, _FM), jnp.float32),
        pltpu.SemaphoreType.DMA,
        pltpu.SemaphoreType.DMA,
        pltpu.SemaphoreType.DMA,
        pltpu.SemaphoreType.DMA,
    ],
)(_segsum_body)


def _mm(x, W, b):
    return jnp.dot(x, W, preferred_element_type=jnp.float32) + b


def _pairnorm(y):
    y = y - jnp.mean(y, axis=0, keepdims=True)
    rn = jnp.sqrt(1e-6 + jnp.mean(jnp.sum(y * y, axis=1)))
    return y / rn


def _q_body(var_ref, qc1W, qc1b, qc2W, qc2b, qo1W, qo1b, qo2W, qo2b, objm,
            cq_ref, oq_ref):
    v = var_ref[...]
    h = jnp.maximum(_mm(v, qc1W[...], qc1b[...]), 0.0)
    cq_ref[...] = _mm(h, qc2W[...], qc2b[...])
    h = jnp.maximum(_mm(v, qo1W[...], qo1b[...]), 0.0)
    oq_ref[...] = _mm(h, qo2W[...], qo2b[...]) * objm[...]


def _c_body(con_ref, v2c_ref, cv_ref, cu1aW, cu1bW, cu1b, cu2W, cu2b,
            out_ref):
    v2c = v2c_ref[:_NCON]
    loss = jnp.maximum(v2c - cv_ref[...], 0.0)
    h = jnp.dot(con_ref[...], cu1aW[...], preferred_element_type=jnp.float32)
    h = h + jnp.dot(loss, cu1bW[...], preferred_element_type=jnp.float32)
    h = jnp.maximum(h + cu1b[...], 0.0)
    out_ref[...] = _pairnorm(_mm(h, cu2W[...], cu2b[...]))


def _v_body(var_ref, c2v_ref, oq_ref, vu1aW, vu1bW, vu1cW, vu1b, vu2W, vu2b,
            o1W, o1b, o2W, o2b, nim_ref, newvar_ref, out_ref):
    h = jnp.dot(var_ref[...], vu1aW[...], preferred_element_type=jnp.float32)
    h = h + jnp.dot(c2v_ref[:_NV], vu1bW[...],
                    preferred_element_type=jnp.float32)
    h = h + jnp.dot(oq_ref[...], vu1cW[...], preferred_element_type=jnp.float32)
    h = jnp.maximum(h + vu1b[...], 0.0)
    y = _pairnorm(_mm(h, vu2W[...], vu2b[...]))
    newvar_ref[...] = y
    h = jnp.maximum(_mm(y, o1W[...], o1b[...]), 0.0)
    out_ref[...] = jax.nn.sigmoid(_mm(h, o2W[...], o2b[...]) + nim_ref[...])


def _f32(shape):
    return jax.ShapeDtypeStruct(shape, jnp.float32)


def _partition_edges(ga, sa, va):
    g2 = ga.reshape(_NSUB, _EPS)
    s2 = sa.reshape(_NSUB, _EPS)
    v2 = va.reshape(_NSUB, _EPS)
    gsp = g2 | (s2 << 14)
    bit = (s2 >= _RPC).astype(jnp.int32)
    pos = jnp.arange(_EPS, dtype=jnp.int32)[None, :]
    gs_out, v_out, n_out = [], [], []
    for core in (0, 1):
        mine = bit if core else 1 - bit
        cnt = jnp.sum(mine, axis=1, dtype=jnp.int32)
        dest = jnp.where(mine == 1, jnp.cumsum(mine, axis=1) - 1,
                         cnt[:, None] + jnp.cumsum(1 - mine, axis=1) - 1)
        gs_c = jnp.put_along_axis(jnp.zeros_like(gsp), dest, gsp, axis=1,
                                  inplace=False)
        v_c = jnp.put_along_axis(jnp.zeros_like(v2), dest, v2, axis=1,
                                 inplace=False)
        valid = pos < cnt[:, None]
        gs_c = jnp.where(valid, gs_c, (core * _RPC) << 14)
        v_c = jnp.where(valid, v_c, 0.0)
        gs_out.append(jnp.pad(gs_c, ((0, 0), (0, _EPSP - _EPS)),
                              constant_values=(core * _RPC) << 14))
        v_out.append(jnp.pad(v_c, ((0, 0), (0, _EPSP - _EPS))))
        n_out.append(jnp.clip((cnt + _CH - 1) // _CH, 2, _NCHUNK))
    nch2 = jnp.stack(n_out).astype(jnp.int32)
    nch3 = jnp.broadcast_to(nch2[:, :, None], (2, _NSUB, 16))
    return jnp.stack(gs_out), jnp.stack(v_out), nch3


def kernel(edge_index, edge_vals, objective_multipliers, const_values,
           integer_mask, cu1_W, cu1_b, cu2_W, cu2_b, qc1_W, qc1_b, qc2_W,
           qc2_b, qo1_W, qo1_b, qo2_W, qo2_b, vu1_W, vu1_b, vu2_W, vu2_b,
           o1_W, o1_b, o2_W, o2_b):
    gs_a, vals_a, cnt_a = _partition_edges(edge_index[0], edge_index[1],
                                           edge_vals)
    gs_b, vals_b, cnt_b = _partition_edges(edge_index[1], edge_index[0],
                                           edge_vals)
    zeros = jnp.zeros((_RPS, _FM), dtype=jnp.float32)
    objm = objective_multipliers[:, None]
    cv = const_values[:, None]
    im = integer_mask[:, None]
    b = {n: v.reshape(1, -1) for n, v in (
        ("cu1", cu1_b), ("cu2", cu2_b), ("qc1", qc1_b), ("qc2", qc2_b),
        ("qo1", qo1_b), ("qo2", qo2_b), ("vu1", vu1_b), ("vu2", vu2_b),
        ("o1", o1_b), ("o2", o2_b))}

    q_call = pl.pallas_call(
        _q_body, out_shape=[_f32((_NV, _FM)), _f32((_NV, _FM))])
    c_call = pl.pallas_call(_c_body, out_shape=_f32((_NCON, _FM)))
    v_call = pl.pallas_call(
        _v_body, out_shape=[_f32((_NV, _FM)), _f32((_NV, _OUT))])

    variables = jnp.ones((_NV, _FM), dtype=jnp.float32)
    constraints = jnp.ones((_NCON, _FM), dtype=jnp.float32)
    nkey = jax.random.key(42)
    outputs = []
    for i in range(3):
        cq, oq = q_call(variables, qc1_W, b["qc1"], qc2_W, b["qc2"],
                        qo1_W, b["qo1"], qo2_W, b["qo2"], objm)
        v2c = _segsum_sc(cq, gs_a, vals_a, cnt_a, zeros)
        constraints = c_call(constraints, v2c, cv, cu1_W[:_FM],
                             cu1_W[_FM:], b["cu1"], cu2_W, b["cu2"])
        c2v = _segsum_sc(constraints, gs_b, vals_b, cnt_b, zeros)
        noise = jax.random.normal(jax.random.fold_in(nkey, i), (_NV, _OUT),
                                  dtype=jnp.float32)
        variables, out_i = v_call(
            variables, c2v, oq, vu1_W[:_FM], vu1_W[_FM:2 * _FM],
            vu1_W[2 * _FM:], b["vu1"], vu2_W, b["vu2"], o1_W, b["o1"],
            o2_W, b["o2"], noise * im)
        outputs.append(out_i)
    return jnp.stack(outputs)


# speedup vs baseline: 5.2089x; 5.2089x over previous
import functools

import jax
import jax.numpy as jnp
from jax import lax
from jax.experimental import pallas as pl
from jax.experimental.pallas import tpu as pltpu
from jax.experimental.pallas import tpu_sc as plsc

_FM = 128
_NV = 10000
_NCON = 10000
_NE = 320000
_OUT = 16

_NSC = 2
_NSUB = 16
_EPS = _NE // _NSUB
_CH = 64
_NCHUNK = 314
_EPSP = _NCHUNK * _CH
_RPC = 5120
_NPAD = _NSC * _RPC
_RPS = _RPC // _NSUB

_DNUMS = lax.GatherDimensionNumbers(
    offset_dims=(), collapsed_slice_dims=(0,), start_index_map=(0,))


def _segsum_body(table, gs, vals, cnts, zeros, out,
                 gs_v, vals_v, cnt_v, gbuf0, gbuf1, sloc0, sloc1,
                 grows0, grows1, srows0, srows1, acc,
                 gsem0, gsem1, ssem0, ssem1):
    c = lax.axis_index("c")
    s = lax.axis_index("s")
    base = c * _RPC
    gbuf = (gbuf0, gbuf1)
    sloc = (sloc0, sloc1)
    grows = (grows0, grows1)
    srows = (srows0, srows1)
    gsem = (gsem0, gsem1)
    ssem = (ssem0, ssem1)

    pltpu.sync_copy(zeros, acc.at[pl.ds(s * _RPS, _RPS)])
    pltpu.sync_copy(gs.at[c, s], gs_v)
    pltpu.sync_copy(vals.at[c, s], vals_v)
    pltpu.sync_copy(cnts.at[c, s], cnt_v)
    nch = cnt_v[pl.ds(0, 16)][0]

    def unpack_to(tt, gb):
        def up(g, carry):
            w16 = gs_v[pl.ds(tt * _CH + g * 16, 16)]
            gb[pl.ds(g * 16, 16)] = jnp.bitwise_and(w16, 0x3FFF)
            return carry
        lax.fori_loop(0, _CH // 16, up, 0)

    unpack_to(0, gbuf0)
    pltpu.async_copy(table.at[gbuf0], grows0, gsem0)
    pltpu.async_copy(zeros.at[pl.ds(0, _CH)], srows0, ssem0)
    pltpu.async_copy(zeros.at[pl.ds(0, _CH)], srows1, ssem1)
    plsc.subcore_barrier()

    def step(t, p):
        q = 1 - p
        tn = jnp.minimum(t + 1, _NCHUNK - 1)

        live_n = t + 1 < nch

        @pl.when(live_n)
        def _prefetch():
            unpack_to(tn, gbuf[q])

        @pl.when(t < nch)
        def _work():
            pltpu.make_async_copy(table.at[pl.ds(0, _CH)], grows[p],
                                  gsem[p]).wait()

            @pl.when(live_n)
            def _g():
                pltpu.async_copy(table.at[gbuf[q]], grows[q], gsem[q])

            pltpu.make_async_copy(zeros.at[pl.ds(0, _CH)], srows[p],
                                  ssem[p]).wait()

            def grp(g, carry2):
                w16 = gs_v[pl.ds(t * _CH + g * 16, 16)]
                sloc[p][pl.ds(g * 16, 16)] = (
                    lax.shift_right_logical(w16, 14) - base)
                v16 = vals_v[pl.ds(t * _CH + g * 16, 16)]
                for j in range(16):
                    b16 = lax.gather(
                        v16, jnp.full((16, 1), j, jnp.int32), _DNUMS, (1,),
                        mode=lax.GatherScatterMode.PROMISE_IN_BOUNDS)
                    r = g * 16 + j
                    for k in range(_FM // 16):
                        srows[p][r, pl.ds(k * 16, 16)] = (
                            grows[p][r, pl.ds(k * 16, 16)] * b16)
                return carry2

            lax.fori_loop(0, _CH // 16, grp, 0)
            pltpu.async_copy(srows[p], acc.at[sloc[p]], ssem[p], add=True)

    def duo(t2, carry):
        step(t2 * 2, 0)
        step(t2 * 2 + 1, 1)
        return carry

    lax.fori_loop(0, _NCHUNK // 2, duo, 0)
    pltpu.make_async_copy(zeros.at[pl.ds(0, _CH)], srows0, ssem0).wait()
    pltpu.make_async_copy(zeros.at[pl.ds(0, _CH)], srows1, ssem1).wait()
    plsc.subcore_barrier()
    pltpu.sync_copy(acc.at[pl.ds(s * _RPS, _RPS)],
                    out.at[pl.ds(base + s * _RPS, _RPS)])


_segsum_sc = functools.partial(
    pl.kernel,
    mesh=plsc.VectorSubcoreMesh(core_axis_name="c", subcore_axis_name="s"),
    out_type=jax.ShapeDtypeStruct((_NPAD, _FM), jnp.float32),
    scratch_types=[
        pltpu.VMEM((_EPSP,), jnp.int32),
        pltpu.VMEM((_EPSP,), jnp.float32),
        pltpu.VMEM((16,), jnp.int32),
        pltpu.VMEM((_CH,), jnp.int32),
        pltpu.VMEM((_CH,), jnp.int32),
        pltpu.VMEM((_CH,), jnp.int32),
        pltpu.VMEM((_CH,), jnp.int32),
        pltpu.VMEM((_CH, _FM), jnp.float32),
        pltpu.VMEM((_CH, _FM), jnp.float32),
        pltpu.VMEM((_CH, _FM), jnp.float32),
        pltpu.VMEM((_CH, _FM), jnp.float32),
        pltpu.VMEM_SHARED((_RPC, _FM), jnp.float32),
        pltpu.SemaphoreType.DMA,
        pltpu.SemaphoreType.DMA,
        pltpu.SemaphoreType.DMA,
        pltpu.SemaphoreType.DMA,
    ],
)(_segsum_body)


def _mm(x, W, b):
    return jnp.dot(x, W, preferred_element_type=jnp.float32) + b


def _pairnorm(y):
    y = y - jnp.mean(y, axis=0, keepdims=True)
    rn = jnp.sqrt(1e-6 + jnp.mean(jnp.sum(y * y, axis=1)))
    return y / rn


def _q_body(var_ref, qc1W, qc1b, qc2W, qc2b, qo1W, qo1b, qo2W, qo2b, objm,
            cq_ref, oq_ref):
    v = var_ref[...]
    h = jnp.maximum(_mm(v, qc1W[...], qc1b[...]), 0.0)
    cq_ref[...] = _mm(h, qc2W[...], qc2b[...])
    h = jnp.maximum(_mm(v, qo1W[...], qo1b[...]), 0.0)
    oq_ref[...] = _mm(h, qo2W[...], qo2b[...]) * objm[...]


def _c_body(con_ref, v2c_ref, cv_ref, cu1aW, cu1bW, cu1b, cu2W, cu2b,
            out_ref):
    v2c = v2c_ref[:_NCON]
    loss = jnp.maximum(v2c - cv_ref[...], 0.0)
    h = jnp.dot(con_ref[...], cu1aW[...], preferred_element_type=jnp.float32)
    h = h + jnp.dot(loss, cu1bW[...], preferred_element_type=jnp.float32)
    h = jnp.maximum(h + cu1b[...], 0.0)
    out_ref[...] = _pairnorm(_mm(h, cu2W[...], cu2b[...]))


def _v_body(var_ref, c2v_ref, oq_ref, vu1aW, vu1bW, vu1cW, vu1b, vu2W, vu2b,
            o1W, o1b, o2W, o2b, nim_ref, newvar_ref, out_ref):
    h = jnp.dot(var_ref[...], vu1aW[...], preferred_element_type=jnp.float32)
    h = h + jnp.dot(c2v_ref[:_NV], vu1bW[...],
                    preferred_element_type=jnp.float32)
    h = h + jnp.dot(oq_ref[...], vu1cW[...], preferred_element_type=jnp.float32)
    h = jnp.maximum(h + vu1b[...], 0.0)
    y = _pairnorm(_mm(h, vu2W[...], vu2b[...]))
    newvar_ref[...] = y
    h = jnp.maximum(_mm(y, o1W[...], o1b[...]), 0.0)
    out_ref[...] = jax.nn.sigmoid(_mm(h, o2W[...], o2b[...]) + nim_ref[...])


def _f32(shape):
    return jax.ShapeDtypeStruct(shape, jnp.float32)


def _partition_edges(ga, sa, va):
    g2 = ga.reshape(_NSUB, _EPS)
    s2 = sa.reshape(_NSUB, _EPS)
    v2 = va.reshape(_NSUB, _EPS)
    gsp = g2 | (s2 << 14)
    bit = (s2 >= _RPC).astype(jnp.int32)
    cnt0 = _EPS - jnp.sum(bit, axis=1, dtype=jnp.int32)
    _, gs_s, v_s = lax.sort((bit, gsp, v2), dimension=1, num_keys=1,
                            is_stable=False)
    pos = jnp.arange(_EPS, dtype=jnp.int32)[None, :]
    gs_out, v_out, n_out = [], [], []
    for core in (0, 1):
        if core == 0:
            gs_c, v_c, cnt = gs_s, v_s, cnt0
        else:
            idx = (pos + cnt0[:, None]) % _EPS
            gs_c = jnp.take_along_axis(gs_s, idx, axis=1)
            v_c = jnp.take_along_axis(v_s, idx, axis=1)
            cnt = _EPS - cnt0
        valid = pos < cnt[:, None]
        gs_c = jnp.where(valid, gs_c, (core * _RPC) << 14)
        v_c = jnp.where(valid, v_c, 0.0)
        gs_out.append(jnp.pad(gs_c, ((0, 0), (0, _EPSP - _EPS)),
                              constant_values=(core * _RPC) << 14))
        v_out.append(jnp.pad(v_c, ((0, 0), (0, _EPSP - _EPS))))
        n_out.append(jnp.clip((cnt + _CH - 1) // _CH, 2, _NCHUNK))
    nch2 = jnp.stack(n_out).astype(jnp.int32)
    nch3 = jnp.broadcast_to(nch2[:, :, None], (2, _NSUB, 16))
    return jnp.stack(gs_out), jnp.stack(v_out), nch3


def kernel(edge_index, edge_vals, objective_multipliers, const_values,
           integer_mask, cu1_W, cu1_b, cu2_W, cu2_b, qc1_W, qc1_b, qc2_W,
           qc2_b, qo1_W, qo1_b, qo2_W, qo2_b, vu1_W, vu1_b, vu2_W, vu2_b,
           o1_W, o1_b, o2_W, o2_b):
    gs_a, vals_a, cnt_a = _partition_edges(edge_index[0], edge_index[1],
                                           edge_vals)
    gs_b, vals_b, cnt_b = _partition_edges(edge_index[1], edge_index[0],
                                           edge_vals)
    zeros = jnp.zeros((_RPS, _FM), dtype=jnp.float32)
    objm = objective_multipliers[:, None]
    cv = const_values[:, None]
    im = integer_mask[:, None]
    b = {n: v.reshape(1, -1) for n, v in (
        ("cu1", cu1_b), ("cu2", cu2_b), ("qc1", qc1_b), ("qc2", qc2_b),
        ("qo1", qo1_b), ("qo2", qo2_b), ("vu1", vu1_b), ("vu2", vu2_b),
        ("o1", o1_b), ("o2", o2_b))}

    q_call = pl.pallas_call(
        _q_body, out_shape=[_f32((_NV, _FM)), _f32((_NV, _FM))])
    c_call = pl.pallas_call(_c_body, out_shape=_f32((_NCON, _FM)))
    v_call = pl.pallas_call(
        _v_body, out_shape=[_f32((_NV, _FM)), _f32((_NV, _OUT))])

    variables = jnp.ones((_NV, _FM), dtype=jnp.float32)
    constraints = jnp.ones((_NCON, _FM), dtype=jnp.float32)
    nkey = jax.random.key(42)
    outputs = []
    for i in range(3):
        cq, oq = q_call(variables, qc1_W, b["qc1"], qc2_W, b["qc2"],
                        qo1_W, b["qo1"], qo2_W, b["qo2"], objm)
        v2c = _segsum_sc(cq, gs_a, vals_a, cnt_a, zeros)
        constraints = c_call(constraints, v2c, cv, cu1_W[:_FM],
                             cu1_W[_FM:], b["cu1"], cu2_W, b["cu2"])
        c2v = _segsum_sc(constraints, gs_b, vals_b, cnt_b, zeros)
        noise = jax.random.normal(jax.random.fold_in(nkey, i), (_NV, _OUT),
                                  dtype=jnp.float32)
        variables, out_i = v_call(
            variables, c2v, oq, vu1_W[:_FM], vu1_W[_FM:2 * _FM],
            vu1_W[2 * _FM:], b["vu1"], vu2_W, b["vu2"], o1_W, b["o1"],
            o2_W, b["o2"], noise * im)
        outputs.append(out_i)
    return jnp.stack(outputs)
